# Initial kernel scaffold; baseline (speedup 1.0000x reference)
import jax
import jax.numpy as jnp
from jax.experimental import pallas as pl


def kernel(pointcloud, params):
    raise NotImplementedError("write your pallas kernel here")


# speedup vs baseline: 3.1346x; 3.1346x over previous
import functools

import jax
import jax.numpy as jnp
from jax.experimental import pallas as pl

_HI = jax.lax.Precision.HIGHEST


def _bdot(a, b):
    return jax.lax.dot(a.astype(jnp.bfloat16), b.astype(jnp.bfloat16),
                       preferred_element_type=jnp.float32)
_NPOINTS = [1024, 256, 64, 16]
_RADIUS = [0.1, 0.2, 0.4, 0.8]
_NSAMPLE = 32
_EPS = 1e-5
_NUM_CLASSES = 13


def _fps_kernel(xs_ref, ys_ref, zs_ref, out_ref, *, npoint):
    xs = xs_ref[0]
    ys = ys_ref[0]
    zs = zs_ref[0]
    r, lanes = xs.shape
    flat = (jax.lax.broadcasted_iota(jnp.int32, (r, lanes), 0) * lanes
            + jax.lax.broadcasted_iota(jnp.int32, (r, lanes), 1))
    rowi = jax.lax.broadcasted_iota(jnp.int32, (npoint, 1), 0)

    def body(s, carry):
        dist, far, nx, ny, nz = carry
        eq = flat == far
        cx = jnp.sum(jnp.where(eq, xs, 0.0))
        cy = jnp.sum(jnp.where(eq, ys, 0.0))
        cz = jnp.sum(jnp.where(eq, zs, 0.0))
        sel = rowi == s
        nx = jnp.where(sel, cx, nx)
        ny = jnp.where(sel, cy, ny)
        nz = jnp.where(sel, cz, nz)
        d = (xs - cx) ** 2 + (ys - cy) ** 2 + (zs - cz) ** 2
        dist = jnp.minimum(dist, d)
        m = jnp.max(dist)
        far = jnp.min(jnp.where(dist == m, flat, jnp.int32(2 ** 30)))
        return dist, far, nx, ny, nz

    init = (jnp.full((r, lanes), 1e10, jnp.float32), jnp.int32(0),
            jnp.zeros((npoint, 1), jnp.float32),
            jnp.zeros((npoint, 1), jnp.float32),
            jnp.zeros((npoint, 1), jnp.float32))
    _, _, nx, ny, nz = jax.lax.fori_loop(0, npoint, body, init)
    out_ref[0] = jnp.concatenate([nx, ny, nz], axis=1)


def _fps(xyz, npoint):
    b, n, _ = xyz.shape
    lanes = 128 if n % 128 == 0 else n
    r = n // lanes
    xs = xyz[..., 0].reshape(b, r, lanes)
    ys = xyz[..., 1].reshape(b, r, lanes)
    zs = xyz[..., 2].reshape(b, r, lanes)
    return pl.pallas_call(
        functools.partial(_fps_kernel, npoint=npoint),
        grid=(b,),
        in_specs=[pl.BlockSpec((1, r, lanes), lambda i: (i, 0, 0))] * 3,
        out_specs=pl.BlockSpec((1, npoint, 3), lambda i: (i, 0, 0)),
        out_shape=jax.ShapeDtypeStruct((b, npoint, 3), jnp.float32),
    )(xs, ys, zs)


def _ballq_kernel(c_ref, p_ref, o_ref, *, r2, n):
    c = c_ref[...]
    p = p_ref[0]
    sq = (jnp.sum(c * c, axis=1, keepdims=True)
          + jnp.sum(p * p, axis=0, keepdims=True)
          - 2.0 * _bdot(c, p))
    col = jax.lax.broadcasted_iota(jnp.int32, sq.shape, 1)
    g = jnp.where(sq <= r2, col, n)
    cols = []
    for _ in range(_NSAMPLE):
        m = jnp.min(g, axis=1, keepdims=True)
        cols.append(m)
        g = jnp.where(g == m, n, g)
    idx = jnp.concatenate(cols, axis=1)
    idx = jnp.where(idx == n, idx[:, 0:1], idx)
    o_ref[...] = jnp.minimum(idx, n - 1)


def _ballq(new_xyz, xyz, radius):
    b, s, _ = new_xyz.shape
    n = xyz.shape[1]
    sb = min(256, s)
    tpb = s // sb
    c = new_xyz.reshape(b * s, 3)
    p = jnp.transpose(xyz, (0, 2, 1))
    return pl.pallas_call(
        functools.partial(_ballq_kernel, r2=radius * radius, n=n),
        grid=(b * tpb,),
        in_specs=[pl.BlockSpec((sb, 3), lambda i: (i, 0)),
                  pl.BlockSpec((1, 3, n), lambda i: (i // tpb, 0, 0))],
        out_specs=pl.BlockSpec((sb, 32), lambda i: (i, 0)),
        out_shape=jax.ShapeDtypeStruct((b * s, 32), jnp.int32),
    )(c, p)


def _gather_mm_kernel(idx_ref, crep_ref, p_ref, w_ref,
                      y_ref, s_ref, q_ref):
    pts = p_ref[0]
    col = jax.lax.broadcasted_iota(
        jnp.int32, (idx_ref.shape[0], pts.shape[0]), 1)
    oh = (idx_ref[...] == col).astype(jnp.float32)
    g = jax.lax.dot(oh, pts, precision=_HI)
    y = _bdot(g - crep_ref[...], w_ref[...])
    y_ref[...] = y

    @pl.when(pl.program_id(0) == 0)
    def _():
        s_ref[...] = jnp.zeros_like(s_ref)
        q_ref[...] = jnp.zeros_like(q_ref)

    s_ref[0:1, :] = s_ref[0:1, :] + jnp.sum(y, axis=0, keepdims=True)
    q_ref[0:1, :] = q_ref[0:1, :] + jnp.sum(y * y, axis=0, keepdims=True)


def _gather_mm(idxf, crep, pts, w1t):
    b, n, c = pts.shape
    rtot = idxf.shape[0]
    gb = min(512, rtot)
    nt = rtot // gb
    tpb = nt // b
    o = w1t.shape[1]
    return pl.pallas_call(
        _gather_mm_kernel,
        grid=(nt,),
        in_specs=[pl.BlockSpec((gb, 1), lambda i: (i, 0)),
                  pl.BlockSpec((gb, c), lambda i: (i, 0)),
                  pl.BlockSpec((1, n, c), lambda i: (i // tpb, 0, 0)),
                  pl.BlockSpec((c, o), lambda i: (0, 0))],
        out_specs=[pl.BlockSpec((gb, o), lambda i: (i, 0)),
                   pl.BlockSpec((8, o), lambda i: (0, 0)),
                   pl.BlockSpec((8, o), lambda i: (0, 0))],
        out_shape=[jax.ShapeDtypeStruct((rtot, o), jnp.float32),
                   jax.ShapeDtypeStruct((8, o), jnp.float32),
                   jax.ShapeDtypeStruct((8, o), jnp.float32)],
    )(idxf, crep, pts, w1t)


def _mm_kernel(x_ref, m_ref, v_ref, g_ref, b_ref, w_ref, y_ref, s_ref, q_ref,
               *, prenorm):
    x = x_ref[...]
    if prenorm:
        x = jnp.maximum((x - m_ref[0:1, :]) / v_ref[0:1, :]
                        * g_ref[0:1, :] + b_ref[0:1, :], 0.0)
    y = _bdot(x, w_ref[...])
    y_ref[...] = y

    @pl.when(pl.program_id(0) == 0)
    def _():
        s_ref[...] = jnp.zeros_like(s_ref)
        q_ref[...] = jnp.zeros_like(q_ref)

    s_ref[0:1, :] = s_ref[0:1, :] + jnp.sum(y, axis=0, keepdims=True)
    q_ref[0:1, :] = q_ref[0:1, :] + jnp.sum(y * y, axis=0, keepdims=True)


def _mm(x, wt, bn=None):
    rtot, cin = x.shape
    o = wt.shape[1]
    rb = min(4096, rtot)
    if bn is None:
        z = jnp.ones((8, cin), jnp.float32)
        bn = (z, z, z, z)
        prenorm = False
    else:
        prenorm = True
    return pl.pallas_call(
        functools.partial(_mm_kernel, prenorm=prenorm),
        grid=(rtot // rb,),
        in_specs=[pl.BlockSpec((rb, cin), lambda i: (i, 0)),
                  pl.BlockSpec((8, cin), lambda i: (0, 0)),
                  pl.BlockSpec((8, cin), lambda i: (0, 0)),
                  pl.BlockSpec((8, cin), lambda i: (0, 0)),
                  pl.BlockSpec((8, cin), lambda i: (0, 0)),
                  pl.BlockSpec((cin, o), lambda i: (0, 0))],
        out_specs=[pl.BlockSpec((rb, o), lambda i: (i, 0)),
                   pl.BlockSpec((8, o), lambda i: (0, 0)),
                   pl.BlockSpec((8, o), lambda i: (0, 0))],
        out_shape=[jax.ShapeDtypeStruct((rtot, o), jnp.float32),
                   jax.ShapeDtypeStruct((8, o), jnp.float32),
                   jax.ShapeDtypeStruct((8, o), jnp.float32)],
    )(x, bn[0], bn[1], bn[2], bn[3], wt)


def _maxfin_kernel(y_ref, m_ref, v_ref, g_ref, b_ref, o_ref):
    m = m_ref[0:1, :][:, None, :]
    v = v_ref[0:1, :][:, None, :]
    g = g_ref[0:1, :][:, None, :]
    b = b_ref[0:1, :][:, None, :]
    x = jnp.maximum((y_ref[...] - m) / v * g + b, 0.0)
    o_ref[...] = jnp.max(x, axis=1)


def _maxfin(y3, bn):
    gtot, ns, o = y3.shape
    gb = min(512, gtot)
    return pl.pallas_call(
        _maxfin_kernel,
        grid=(gtot // gb,),
        in_specs=[pl.BlockSpec((gb, ns, o), lambda i: (i, 0, 0)),
                  pl.BlockSpec((8, o), lambda i: (0, 0)),
                  pl.BlockSpec((8, o), lambda i: (0, 0)),
                  pl.BlockSpec((8, o), lambda i: (0, 0)),
                  pl.BlockSpec((8, o), lambda i: (0, 0))],
        out_specs=pl.BlockSpec((gb, o), lambda i: (i, 0)),
        out_shape=jax.ShapeDtypeStruct((gtot, o), jnp.float32),
    )(y3, bn[0], bn[1], bn[2], bn[3])


def _fin_kernel(x_ref, m_ref, v_ref, g_ref, b_ref, o_ref, *, relu):
    y = ((x_ref[...] - m_ref[0:1, :]) / v_ref[0:1, :]
         * g_ref[0:1, :] + b_ref[0:1, :])
    if relu:
        y = jnp.maximum(y, 0.0)
    o_ref[...] = y


def _fin(x, bn, relu):
    rtot, ch = x.shape
    rb = min(4096, rtot)
    return pl.pallas_call(
        functools.partial(_fin_kernel, relu=relu),
        grid=(rtot // rb,),
        in_specs=[pl.BlockSpec((rb, ch), lambda i: (i, 0)),
                  pl.BlockSpec((8, ch), lambda i: (0, 0)),
                  pl.BlockSpec((8, ch), lambda i: (0, 0)),
                  pl.BlockSpec((8, ch), lambda i: (0, 0)),
                  pl.BlockSpec((8, ch), lambda i: (0, 0))],
        out_specs=pl.BlockSpec((rb, ch), lambda i: (i, 0)),
        out_shape=jax.ShapeDtypeStruct((rtot, ch), jnp.float32),
    )(x, bn[0], bn[1], bn[2], bn[3])


def _interp_kernel(c_ref, p2t_ref, f2_ref, f1_ref, wa_ref, wb_ref,
                   y_ref, s_ref, q_ref, *, s2):
    c1 = c_ref[...]
    pt = p2t_ref[0]
    sq = (jnp.sum(c1 * c1, axis=1, keepdims=True)
          + jnp.sum(pt * pt, axis=0, keepdims=True)
          - 2.0 * _bdot(c1, pt))
    col = jax.lax.broadcasted_iota(jnp.int32, sq.shape, 1)
    ws = []
    gs = []
    for _ in range(3):
        m = jnp.min(sq, axis=1, keepdims=True)
        am = jnp.min(jnp.where(sq == m, col, s2), axis=1, keepdims=True)
        ws.append(1.0 / (jnp.maximum(m, 0.0) + 1e-8))
        oh = (col == am).astype(jnp.float32)
        gs.append(jax.lax.dot(oh, f2_ref[0], precision=_HI))
        sq = jnp.where(col == am, jnp.float32(3e38), sq)
    wsum = (ws[0] + ws[1]) + ws[2]
    interp = ((ws[0] / wsum) * gs[0] + (ws[1] / wsum) * gs[1]) \
        + (ws[2] / wsum) * gs[2]
    y = _bdot(f1_ref[...], wa_ref[...]) + _bdot(interp, wb_ref[...])
    y_ref[...] = y

    @pl.when(pl.program_id(0) == 0)
    def _():
        s_ref[...] = jnp.zeros_like(s_ref)
        q_ref[...] = jnp.zeros_like(q_ref)

    s_ref[0:1, :] = s_ref[0:1, :] + jnp.sum(y, axis=0, keepdims=True)
    q_ref[0:1, :] = q_ref[0:1, :] + jnp.sum(y * y, axis=0, keepdims=True)


def _interp_mm(xyz1, xyz2t, f1, f2, wa, wb):
    b, n1, _ = xyz1.shape
    s2 = xyz2t.shape[2]
    c1 = f1.shape[1]
    c2 = f2.shape[2]
    o = wa.shape[1]
    nb = min(512, n1)
    tpb = n1 // nb
    rtot = b * n1
    return pl.pallas_call(
        functools.partial(_interp_kernel, s2=s2),
        grid=(b * tpb,),
        in_specs=[pl.BlockSpec((nb, 3), lambda i: (i, 0)),
                  pl.BlockSpec((1, 3, s2), lambda i: (i // tpb, 0, 0)),
                  pl.BlockSpec((1, s2, c2), lambda i: (i // tpb, 0, 0)),
                  pl.BlockSpec((nb, c1), lambda i: (i, 0)),
                  pl.BlockSpec((c1, o), lambda i: (0, 0)),
                  pl.BlockSpec((c2, o), lambda i: (0, 0))],
        out_specs=[pl.BlockSpec((nb, o), lambda i: (i, 0)),
                   pl.BlockSpec((8, o), lambda i: (0, 0)),
                   pl.BlockSpec((8, o), lambda i: (0, 0))],
        out_shape=[jax.ShapeDtypeStruct((rtot, o), jnp.float32),
                   jax.ShapeDtypeStruct((8, o), jnp.float32),
                   jax.ShapeDtypeStruct((8, o), jnp.float32)],
    )(xyz1.reshape(rtot, 3), xyz2t, f2, f1, wa, wb)


def _bn_ac(ssum, ssq, cnt, g, b):
    mean = ssum[0] / cnt
    var = ssq[0] / cnt - mean * mean
    sv = jnp.sqrt(var + _EPS)
    ch = mean.shape[0]
    return (jnp.broadcast_to(mean, (8, ch)), jnp.broadcast_to(sv, (8, ch)),
            jnp.broadcast_to(g, (8, ch)), jnp.broadcast_to(b, (8, ch)))


def _bn_ref(y, shape, axes, g, b):
    y4 = y.reshape(shape)
    mean = jnp.mean(y4, axis=axes)
    var = jnp.var(y4, axis=axes)
    sv = jnp.sqrt(var + _EPS)
    ch = mean.shape[0]
    return (jnp.broadcast_to(mean, (8, ch)), jnp.broadcast_to(sv, (8, ch)),
            jnp.broadcast_to(g, (8, ch)), jnp.broadcast_to(b, (8, ch)))


def kernel(pointcloud, params):
    b, n, _ = pointcloud.shape
    xyz = pointcloud[..., 0:3]
    feats = pointcloud[..., 3:]
    l_xyz = [xyz]
    l_feat = [feats]
    for k in range(4):
        s = _NPOINTS[k]
        x_ = l_xyz[k]
        f_ = l_feat[k]
        nxyz = _fps(x_, s)
        idx = _ballq(nxyz, x_, _RADIUS[k])
        pts = jnp.concatenate([x_, f_], axis=-1)
        idxf = idx.reshape(b * s * _NSAMPLE, 1)
        crep3 = jnp.repeat(nxyz.reshape(b * s, 3), _NSAMPLE, axis=0)
        crep = jnp.concatenate(
            [crep3, jnp.zeros((crep3.shape[0], pts.shape[2] - 3),
                              jnp.float32)], axis=1)
        lay = [(w.T, g_, b_) for (w, g_, b_) in params['sa'][k]]
        cnt = b * s * _NSAMPLE
        y1, s1, q1 = _gather_mm(idxf, crep, pts, lay[0][0])
        sh = (b, s, _NSAMPLE, -1)
        bn1 = _bn_ref(y1, sh, (0, 1, 2), lay[0][1], lay[0][2])
        y2, s2_, q2 = _mm(y1, lay[1][0], bn1)
        bn2 = _bn_ref(y2, sh, (0, 1, 2), lay[1][1], lay[1][2])
        y3, s3, q3 = _mm(y2, lay[2][0], bn2)
        bn3 = _bn_ref(y3, sh, (0, 1, 2), lay[2][1], lay[2][2])
        o3 = y3.shape[1]
        out = _maxfin(y3.reshape(b * s, _NSAMPLE, o3), bn3)
        l_xyz.append(nxyz)
        l_feat.append(out.reshape(b, s, o3))
    for i in range(-1, -5, -1):
        xyz1 = l_xyz[i - 1]
        xyz2 = l_xyz[i]
        p1 = l_feat[i - 1]
        p2 = l_feat[i]
        layers = params['fp'][i]
        n1 = xyz1.shape[1]
        c1ch = p1.shape[2]
        w1t = layers[0][0].T
        cnt = b * n1
        y, su, qu = _interp_mm(xyz1, jnp.transpose(xyz2, (0, 2, 1)),
                               p1.reshape(b * n1, c1ch), p2,
                               w1t[:c1ch], w1t[c1ch:])
        shf = (b, n1, -1)
        bnf = _bn_ref(y, shf, (0, 1), layers[0][1], layers[0][2])
        y2_, s2b, q2b = _mm(y, layers[1][0].T, bnf)
        bnf2 = _bn_ref(y2_, shf, (0, 1), layers[1][1], layers[1][2])
        newf = _fin(y2_, bnf2, relu=True)
        l_feat[i - 1] = newf.reshape(b, n1, newf.shape[1])
    x0 = l_feat[0].reshape(b * n, l_feat[0].shape[2])
    (w0, g0, b0), (w1, g1, b1) = params['cls']
    yc0, sc0, qc0 = _mm(x0, w0.T)
    bnc0 = _bn_ref(yc0, (b, n, -1), (0, 1), g0, b0)
    yc1, sc1, qc1 = _mm(yc0, w1.T, bnc0)
    bnc1 = _bn_ref(yc1, (b, n, -1), (0, 1), g1, b1)
    out = _fin(yc1, bnc1, relu=False)
    return out.reshape(b, n, _NUM_CLASSES)
